# baseline (speedup 1.0000x reference)
import jax
import jax.numpy as jnp
from jax.experimental import pallas as pl

NUM_LAYER = 8
NUM_PATH = 4
NUM_PATH_NODE = 40000
NUM_EDGE_NODE = 10000
N = NUM_EDGE_NODE + NUM_PATH_NODE
E = 800000


def _gru_cell(x, h, p):
    gi = x @ p["W_ih"].T + p["b_ih"]
    gh = h @ p["W_hh"].T + p["b_hh"]
    i_r, i_z, i_n = jnp.split(gi, 3, axis=1)
    h_r, h_z, h_n = jnp.split(gh, 3, axis=1)
    r = jax.nn.sigmoid(i_r + h_r)
    z = jax.nn.sigmoid(i_z + h_z)
    n = jnp.tanh(i_n + r * h_n)
    return (1.0 - z) * n + z * h


def _spmm(row, col, vals, h):
    return jnp.zeros((N, h.shape[1]), h.dtype).at[row].add(vals[:, None] * h[col])


def _copy_kernel(x_ref, o_ref):
    o_ref[...] = x_ref[...]


def kernel(h_0, edge_index, edge_values, params):
    row, col = edge_index[0], edge_index[1]
    h_i = h_0
    for i in range(NUM_LAYER):
        msg = _spmm(row, col, edge_values, h_i)
        h_path_new = _gru_cell(msg[-NUM_PATH_NODE:, :], h_i[-NUM_PATH_NODE:, :],
                               params["path_gru"][i])
        msg_t = _spmm(col, row, edge_values, h_i)
        h_edge_new = _gru_cell(msg_t[:-NUM_PATH_NODE, :], h_i[:-NUM_PATH_NODE, :],
                               params["edge_gru"][i])
        d = h_path_new.shape[1]
        p = params["dnn"][i]
        hp = h_path_new.reshape(NUM_PATH_NODE // NUM_PATH, NUM_PATH * d) @ p["W"].T + p["b"]
        hp = hp.reshape(NUM_PATH_NODE, d)
        h_i = jnp.concatenate([h_edge_new, hp], axis=0)
        h_i = jnp.concatenate([h_i, h_0], axis=-1)
    out = h_i[-NUM_PATH_NODE:, :]
    return pl.pallas_call(
        _copy_kernel,
        out_shape=jax.ShapeDtypeStruct(out.shape, out.dtype),
    )(out)


# speedup vs baseline: 21.7406x; 21.7406x over previous
import functools

import jax
import jax.numpy as jnp
from jax import lax
from jax.experimental import pallas as pl
from jax.experimental.pallas import tpu as pltpu
from jax.experimental.pallas import tpu_sc as plsc

NUM_LAYER = 8
NUM_PATH = 4
NUM_PATH_NODE = 40000
NUM_EDGE_NODE = 10000
N = NUM_EDGE_NODE + NUM_PATH_NODE
E = 800000

NW = 32
EPT = E // NW
K = 128
ROWS = (EPT + K - 1) // K
EPT_PAD = ROWS * K
G = 4


@functools.lru_cache(maxsize=None)
def _make_spmm_kernel(d):
    q2 = 2 * d
    mesh = plsc.VectorSubcoreMesh(core_axis_name="c", subcore_axis_name="s")

    @functools.partial(
        pl.kernel,
        out_type=jax.ShapeDtypeStruct((2, q2, N), jnp.float32),
        mesh=mesh,
        compiler_params=pltpu.CompilerParams(
            needs_layout_passes=False, use_tc_tiling_on_sc=False),
        scratch_types=[
            pltpu.VMEM((N,), jnp.float32),
            pltpu.VMEM((G, K), jnp.int32),
            pltpu.VMEM((G, K), jnp.int32),
            pltpu.VMEM((G * K,), jnp.float32),
            pltpu.VMEM((2, G, K), jnp.float32),
            pltpu.VMEM_SHARED((q2, N), jnp.float32),
            pltpu.SemaphoreType.DMA,
            pltpu.SemaphoreType.DMA,
        ],
    )
    def spmm(h_hbm, row_hbm, col_hbm, vals_hbm, zeros_hbm, out_hbm,
             h_j, row_b, col_b, vals_t, upd, acc, sem_g, sem_s):
        cid = lax.axis_index("c")
        sid = lax.axis_index("s")
        wid = cid * 16 + sid

        @pl.when(sid < q2)
        def _():
            pltpu.sync_copy(zeros_hbm, acc.at[sid])

        plsc.subcore_barrier()

        for j in range(d):
            pltpu.sync_copy(h_hbm.at[j], h_j)

            @pl.loop(0, ROWS, step=G)
            def _(g):
                eh = [
                    pltpu.async_copy(row_hbm.at[wid, pl.ds(g, G)], row_b,
                                     sem_g),
                    pltpu.async_copy(col_hbm.at[wid, pl.ds(g, G)], col_b,
                                     sem_g),
                    pltpu.async_copy(vals_hbm.at[wid, pl.ds(g * K, G * K)],
                                     vals_t, sem_g),
                ]
                for h in eh:
                    h.wait()
                sh = []
                for t in range(G):
                    for m in range(K // 16):
                        sl = pl.ds(m * 16, 16)
                        v = vals_t[pl.ds(t * K + m * 16, 16)]
                        g0 = plsc.load_gather(h_j, [col_b[t, sl]])
                        upd[0, t, sl] = g0 * v
                        g1 = plsc.load_gather(h_j, [row_b[t, sl]])
                        upd[1, t, sl] = g1 * v
                    sh.append(pltpu.async_copy(
                        upd.at[0, t], acc.at[j].at[row_b.at[t]], sem_s,
                        add=True))
                    sh.append(pltpu.async_copy(
                        upd.at[1, t], acc.at[d + j].at[col_b.at[t]], sem_s,
                        add=True))
                for h in sh:
                    h.wait()

        plsc.subcore_barrier()

        @pl.when(sid < q2)
        def _():
            pltpu.sync_copy(acc.at[sid], out_hbm.at[cid, sid])

    return spmm


def _gru_cell_t(x, h, p):
    gi = p["W_ih"] @ x + p["b_ih"][:, None]
    gh = p["W_hh"] @ h + p["b_hh"][:, None]
    i_r, i_z, i_n = jnp.split(gi, 3, axis=0)
    h_r, h_z, h_n = jnp.split(gh, 3, axis=0)
    r = jax.nn.sigmoid(i_r + h_r)
    z = jax.nn.sigmoid(i_z + h_z)
    n = jnp.tanh(i_n + r * h_n)
    return (1.0 - z) * n + z * h


def kernel(h_0, edge_index, edge_values, params):
    row, col = edge_index[0], edge_index[1]
    pad = EPT_PAD - EPT
    row_p = jnp.pad(row.reshape(NW, EPT), ((0, 0), (0, pad))).reshape(
        NW, ROWS, K)
    col_p = jnp.pad(col.reshape(NW, EPT), ((0, 0), (0, pad))).reshape(
        NW, ROWS, K)
    vals_p = jnp.pad(edge_values.reshape(NW, EPT), ((0, 0), (0, pad)))
    zeros = jnp.zeros((N,), jnp.float32)

    h0_t = h_0.T
    h_t = h0_t
    for i in range(NUM_LAYER):
        d = i + 1
        out = _make_spmm_kernel(d)(h_t, row_p, col_p, vals_p, zeros)
        s = out[0] + out[1]
        msg_t, msgt_t = s[:d], s[d:]

        hpath = _gru_cell_t(msg_t[:, NUM_EDGE_NODE:], h_t[:, NUM_EDGE_NODE:],
                            params["path_gru"][i])
        hedge = _gru_cell_t(msgt_t[:, :NUM_EDGE_NODE], h_t[:, :NUM_EDGE_NODE],
                            params["edge_gru"][i])

        pdnn = params["dnn"][i]
        ngrp = NUM_PATH_NODE // NUM_PATH
        x4 = hpath.reshape(d, ngrp, NUM_PATH).transpose(2, 0, 1).reshape(
            NUM_PATH * d, ngrp)
        hp4 = pdnn["W"] @ x4 + pdnn["b"][:, None]
        hp = hp4.reshape(NUM_PATH, d, ngrp).transpose(1, 2, 0).reshape(
            d, NUM_PATH_NODE)

        h_t = jnp.concatenate(
            [jnp.concatenate([hedge, hp], axis=1), h0_t], axis=0)
    return h_t[:, NUM_EDGE_NODE:].T


# speedup vs baseline: 25.4753x; 1.1718x over previous
import functools

import jax
import jax.numpy as jnp
from jax import lax
from jax.experimental import pallas as pl
from jax.experimental.pallas import tpu as pltpu
from jax.experimental.pallas import tpu_sc as plsc

NUM_LAYER = 8
NUM_PATH = 4
NUM_PATH_NODE = 40000
NUM_EDGE_NODE = 10000
N = NUM_EDGE_NODE + NUM_PATH_NODE
E = 800000

NW = 32
EPT = E // NW
K = 128
ROWS = (EPT + K - 1) // K
EPT_PAD = ROWS * K
G = 4


@functools.lru_cache(maxsize=None)
def _make_spmm_kernel(d):
    q2 = 2 * d
    mesh = plsc.VectorSubcoreMesh(core_axis_name="c", subcore_axis_name="s")

    @functools.partial(
        pl.kernel,
        out_type=jax.ShapeDtypeStruct((2, q2, N), jnp.float32),
        mesh=mesh,
        compiler_params=pltpu.CompilerParams(
            needs_layout_passes=False, use_tc_tiling_on_sc=False),
        scratch_types=[
            pltpu.VMEM((N,), jnp.float32),
            pltpu.VMEM((G, K), jnp.int32),
            pltpu.VMEM((G, K), jnp.int32),
            pltpu.VMEM((G * K,), jnp.float32),
            pltpu.VMEM((2, G, K), jnp.float32),
            pltpu.VMEM_SHARED((q2, N), jnp.float32),
            pltpu.SemaphoreType.DMA,
            pltpu.SemaphoreType.DMA,
        ],
    )
    def spmm(h_hbm, row_hbm, col_hbm, vals_hbm, zeros_hbm, out_hbm,
             h_j, row_b, col_b, vals_t, upd, acc, sem_g, sem_s):
        cid = lax.axis_index("c")
        sid = lax.axis_index("s")
        wid = cid * 16 + sid

        @pl.when(sid < q2)
        def _():
            pltpu.sync_copy(zeros_hbm, acc.at[sid])

        plsc.subcore_barrier()

        for j in range(d):
            pltpu.sync_copy(h_hbm.at[j], h_j)

            @pl.loop(0, ROWS, step=G)
            def _(g):
                eh = [
                    pltpu.async_copy(row_hbm.at[wid, pl.ds(g, G)], row_b,
                                     sem_g),
                    pltpu.async_copy(col_hbm.at[wid, pl.ds(g, G)], col_b,
                                     sem_g),
                    pltpu.async_copy(vals_hbm.at[wid, pl.ds(g * K, G * K)],
                                     vals_t, sem_g),
                ]
                for h in eh:
                    h.wait()
                sh = []
                for t in range(G):
                    for m in range(K // 16):
                        sl = pl.ds(m * 16, 16)
                        v = vals_t[pl.ds(t * K + m * 16, 16)]
                        g0 = plsc.load_gather(h_j, [col_b[t, sl]])
                        upd[0, t, sl] = g0 * v
                        g1 = plsc.load_gather(h_j, [row_b[t, sl]])
                        upd[1, t, sl] = g1 * v
                    sh.append(pltpu.async_copy(
                        upd.at[0, t], acc.at[j].at[row_b.at[t]], sem_s,
                        add=True))
                    sh.append(pltpu.async_copy(
                        upd.at[1, t], acc.at[d + j].at[col_b.at[t]], sem_s,
                        add=True))
                for h in sh:
                    h.wait()

        plsc.subcore_barrier()

        @pl.when(sid < q2)
        def _():
            pltpu.sync_copy(acc.at[sid], out_hbm.at[cid, sid])

    return spmm


def _gru_cell_t(x, h, p):
    gi = p["W_ih"] @ x + p["b_ih"][:, None]
    gh = p["W_hh"] @ h + p["b_hh"][:, None]
    i_r, i_z, i_n = jnp.split(gi, 3, axis=0)
    h_r, h_z, h_n = jnp.split(gh, 3, axis=0)
    r = jax.nn.sigmoid(i_r + h_r)
    z = jax.nn.sigmoid(i_z + h_z)
    n = jnp.tanh(i_n + r * h_n)
    return (1.0 - z) * n + z * h


def kernel(h_0, edge_index, edge_values, params):
    row, col = edge_index[0], edge_index[1]
    pad = EPT_PAD - EPT
    row_p = jnp.pad(row.reshape(NW, EPT), ((0, 0), (0, pad))).reshape(
        NW, ROWS, K)
    col_p = jnp.pad(col.reshape(NW, EPT), ((0, 0), (0, pad))).reshape(
        NW, ROWS, K)
    vals_p = jnp.pad(edge_values.reshape(NW, EPT), ((0, 0), (0, pad)))
    zeros = jnp.zeros((N,), jnp.float32)

    h0_t = h_0.T
    h_t = h0_t
    msg0 = msgt0 = None
    for i in range(NUM_LAYER):
        d = i + 1
        dc = d if i == 0 else d - 1
        out = _make_spmm_kernel(dc)(h_t[:dc], row_p, col_p, vals_p, zeros)
        s = out[0] + out[1]
        if i == 0:
            msg_t, msgt_t = s[:1], s[1:]
            msg0, msgt0 = msg_t, msgt_t
        else:
            msg_t = jnp.concatenate([s[:dc], msg0], axis=0)
            msgt_t = jnp.concatenate([s[dc:], msgt0], axis=0)

        hpath = _gru_cell_t(msg_t[:, NUM_EDGE_NODE:], h_t[:, NUM_EDGE_NODE:],
                            params["path_gru"][i])
        hedge = _gru_cell_t(msgt_t[:, :NUM_EDGE_NODE], h_t[:, :NUM_EDGE_NODE],
                            params["edge_gru"][i])

        pdnn = params["dnn"][i]
        ngrp = NUM_PATH_NODE // NUM_PATH
        x4 = hpath.reshape(d, ngrp, NUM_PATH).transpose(2, 0, 1).reshape(
            NUM_PATH * d, ngrp)
        hp4 = pdnn["W"] @ x4 + pdnn["b"][:, None]
        hp = hp4.reshape(NUM_PATH, d, ngrp).transpose(1, 2, 0).reshape(
            d, NUM_PATH_NODE)

        h_t = jnp.concatenate(
            [jnp.concatenate([hedge, hp], axis=1), h0_t], axis=0)
    return h_t[:, NUM_EDGE_NODE:].T
